# Initial kernel scaffold; baseline (speedup 1.0000x reference)
import jax
import jax.numpy as jnp
from jax.experimental import pallas as pl


def kernel(X, table, W_i2h, W_h2o, b_h2o):
    raise NotImplementedError("write your pallas kernel here")


# speedup vs baseline: 2.4075x; 2.4075x over previous
import functools

import jax
import jax.numpy as jnp
from jax import lax
from jax.experimental import pallas as pl
from jax.experimental.pallas import tpu as pltpu
from jax.experimental.pallas import tpu_sc as plsc

VOCAB = 1000000
EMB = 64
HID = 100
NCLS = 10
BATCH = 16384
SEQ = 50

NC = 2
NS = 16
NW = NC * NS
LANES = 16

BAGS_PER_W = BATCH // NW
BAGS_PER_STEP = 16
STEPS = BAGS_PER_W // BAGS_PER_STEP
BAGS_PER_WIN = 2
WIN_PER_STEP = BAGS_PER_STEP // BAGS_PER_WIN
WIN_IDX = BAGS_PER_WIN * SEQ


def _sc_bag_mean(x_idx, table):
    mesh = plsc.VectorSubcoreMesh(core_axis_name="c", subcore_axis_name="s")

    @functools.partial(
        pl.kernel,
        out_type=jax.ShapeDtypeStruct((BATCH, EMB), jnp.float32),
        mesh=mesh,
        scratch_types=[
            pltpu.VMEM((WIN_PER_STEP, WIN_IDX), jnp.int32),
            pltpu.VMEM((WIN_PER_STEP, WIN_IDX, EMB), jnp.float32),
            pltpu.VMEM((BAGS_PER_STEP, EMB), jnp.float32),
            pltpu.SemaphoreType.DMA,
        ],
        compiler_params=pltpu.CompilerParams(use_tc_tiling_on_sc=False),
    )
    def sc_kernel(x_hbm, table_hbm, out_hbm, idx_v, rows_v, out_v, sem):
        wid = lax.axis_index("s") * NC + lax.axis_index("c")
        base_bag = wid * BAGS_PER_W

        @pl.loop(0, STEPS)
        def _(step):
            bag0 = pl.multiple_of(base_bag + step * BAGS_PER_STEP, BAGS_PER_STEP)
            irow0 = pl.multiple_of(bag0 * SEQ // WIN_IDX, WIN_PER_STEP)
            pltpu.sync_copy(x_hbm.at[pl.ds(irow0, WIN_PER_STEP)], idx_v)
            copies = [
                pltpu.async_copy(table_hbm.at[idx_v.at[j]], rows_v.at[j], sem)
                for j in range(WIN_PER_STEP)
            ]
            for c in copies:
                c.wait()
            for j in range(WIN_PER_STEP):
                for cbag in range(BAGS_PER_WIN):
                    r0 = cbag * SEQ

                    def body(r, acc):
                        return tuple(
                            acc[d] + rows_v[j, r0 + r, pl.ds(d * LANES, LANES)]
                            for d in range(EMB // LANES)
                        )

                    init = tuple(
                        rows_v[j, r0, pl.ds(d * LANES, LANES)]
                        for d in range(EMB // LANES)
                    )
                    acc = lax.fori_loop(1, SEQ, body, init)
                    ob = j * BAGS_PER_WIN + cbag
                    for d in range(EMB // LANES):
                        out_v[ob, pl.ds(d * LANES, LANES)] = acc[d] * (1.0 / SEQ)
            pltpu.sync_copy(out_v, out_hbm.at[pl.ds(bag0, BAGS_PER_STEP)])

    return sc_kernel(x_idx, table)


def _mlp_body(x_ref, wi_ref, wo_ref, b_ref, o_ref):
    x = x_ref[...]
    h = lax.dot_general(x, wi_ref[...], (((1,), (1,)), ((), ())),
                        preferred_element_type=jnp.float32)
    logits = lax.dot_general(h, wo_ref[...], (((1,), (1,)), ((), ())),
                             preferred_element_type=jnp.float32) + b_ref[...]
    m = jnp.max(logits, axis=1, keepdims=True)
    e = jnp.exp(logits - m)
    o_ref[...] = e / jnp.sum(e, axis=1, keepdims=True)


def _tc_mlp(embs, W_i2h, W_h2o, b_h2o):
    BLK = 2048
    return pl.pallas_call(
        _mlp_body,
        grid=(BATCH // BLK,),
        in_specs=[
            pl.BlockSpec((BLK, EMB), lambda i: (i, 0)),
            pl.BlockSpec((HID, EMB), lambda i: (0, 0)),
            pl.BlockSpec((NCLS, HID), lambda i: (0, 0)),
            pl.BlockSpec((1, NCLS), lambda i: (0, 0)),
        ],
        out_specs=pl.BlockSpec((BLK, NCLS), lambda i: (i, 0)),
        out_shape=jax.ShapeDtypeStruct((BATCH, NCLS), jnp.float32),
    )(embs, W_i2h, W_h2o, b_h2o)


@jax.jit
def kernel(X, table, W_i2h, W_h2o, b_h2o):
    x_idx = X.reshape(BATCH * SEQ // WIN_IDX, WIN_IDX).astype(jnp.int32)
    embs = _sc_bag_mean(x_idx, table)
    return _tc_mlp(embs, W_i2h, W_h2o, b_h2o.reshape(1, NCLS))


# speedup vs baseline: 2.6463x; 1.0992x over previous
import functools

import jax
import jax.numpy as jnp
from jax import lax
from jax.experimental import pallas as pl
from jax.experimental.pallas import tpu as pltpu
from jax.experimental.pallas import tpu_sc as plsc

VOCAB = 1000000
EMB = 64
HID = 100
NCLS = 10
BATCH = 16384
SEQ = 50

NC = 2
NS = 16
NW = NC * NS
LANES = 16

BAGS_PER_W = BATCH // NW
BAGS_PER_STEP = 16
STEPS = BAGS_PER_W // BAGS_PER_STEP
BAGS_PER_WIN = 2
WIN_PER_STEP = BAGS_PER_STEP // BAGS_PER_WIN
WIN_IDX = BAGS_PER_WIN * SEQ


def _sc_bag_mean(x_idx, table):
    mesh = plsc.VectorSubcoreMesh(core_axis_name="c", subcore_axis_name="s")

    @functools.partial(
        pl.kernel,
        out_type=jax.ShapeDtypeStruct((BATCH, EMB), jnp.float32),
        mesh=mesh,
        scratch_types=[
            pltpu.VMEM((2, WIN_PER_STEP, WIN_IDX), jnp.int32),
            pltpu.VMEM((2, WIN_PER_STEP, WIN_IDX, EMB), jnp.float32),
            pltpu.VMEM((2, BAGS_PER_STEP, EMB), jnp.float32),
            pltpu.SemaphoreType.DMA,
            pltpu.SemaphoreType.DMA,
        ],
        compiler_params=pltpu.CompilerParams(use_tc_tiling_on_sc=False),
    )
    def sc_kernel(x_hbm, table_hbm, out_hbm, idx_v, rows_v, out_v, gsem, osem):
        wid = lax.axis_index("s") * NC + lax.axis_index("c")
        base_bag = wid * BAGS_PER_W

        def step_bag0(step):
            return pl.multiple_of(base_bag + step * BAGS_PER_STEP,
                                  BAGS_PER_STEP)

        def fire(step, b):
            irow0 = pl.multiple_of(step_bag0(step) * SEQ // WIN_IDX,
                                   WIN_PER_STEP)
            pltpu.sync_copy(x_hbm.at[pl.ds(irow0, WIN_PER_STEP)], idx_v.at[b])
            for j in range(WIN_PER_STEP):
                pltpu.async_copy(table_hbm.at[idx_v.at[b].at[j]],
                                 rows_v.at[b].at[j], gsem)

        def drain_gathers(b):
            for j in range(WIN_PER_STEP):
                pltpu.make_async_copy(table_hbm.at[idx_v.at[b].at[j]],
                                      rows_v.at[b].at[j], gsem).wait()

        def drain_out(b):
            pltpu.make_async_copy(
                out_v.at[b], out_hbm.at[pl.ds(0, BAGS_PER_STEP)], osem).wait()

        fire(0, 0)

        @pl.loop(0, STEPS // 2)
        def _(s2):
            for b in range(2):
                step = s2 * 2 + b
                drain_gathers(b)

                @pl.when(step + 1 < STEPS)
                def _():
                    fire(step + 1, 1 - b)

                @pl.when(step >= 2)
                def _():
                    drain_out(b)

                for j in range(WIN_PER_STEP):
                    for cbag in range(BAGS_PER_WIN):
                        r0 = cbag * SEQ

                        def body(t, acc):
                            r = t * 10
                            for rr in range(10):
                                acc = tuple(
                                    acc[d] + rows_v[b, j, r0 + r + rr,
                                                    pl.ds(d * LANES, LANES)]
                                    for d in range(EMB // LANES)
                                )
                            return acc

                        zero = jnp.zeros((LANES,), jnp.float32)
                        acc = lax.fori_loop(0, SEQ // 10, body,
                                            (zero,) * (EMB // LANES))
                        ob = j * BAGS_PER_WIN + cbag
                        for d in range(EMB // LANES):
                            out_v[b, ob, pl.ds(d * LANES, LANES)] = (
                                acc[d] * (1.0 / SEQ))
                pltpu.async_copy(out_v.at[b],
                                 out_hbm.at[pl.ds(step_bag0(step),
                                                  BAGS_PER_STEP)], osem)

        drain_out(0)
        drain_out(1)

    return sc_kernel(x_idx, table)


def _mlp_body(x_ref, wi_ref, wo_ref, b_ref, o_ref):
    x = x_ref[...]
    h = lax.dot_general(x, wi_ref[...], (((1,), (1,)), ((), ())),
                        preferred_element_type=jnp.float32)
    logits = lax.dot_general(h, wo_ref[...], (((1,), (1,)), ((), ())),
                             preferred_element_type=jnp.float32) + b_ref[...]
    m = jnp.max(logits, axis=1, keepdims=True)
    e = jnp.exp(logits - m)
    o_ref[...] = e / jnp.sum(e, axis=1, keepdims=True)


def _tc_mlp(embs, W_i2h, W_h2o, b_h2o):
    BLK = 2048
    return pl.pallas_call(
        _mlp_body,
        grid=(BATCH // BLK,),
        in_specs=[
            pl.BlockSpec((BLK, EMB), lambda i: (i, 0)),
            pl.BlockSpec((HID, EMB), lambda i: (0, 0)),
            pl.BlockSpec((NCLS, HID), lambda i: (0, 0)),
            pl.BlockSpec((1, NCLS), lambda i: (0, 0)),
        ],
        out_specs=pl.BlockSpec((BLK, NCLS), lambda i: (i, 0)),
        out_shape=jax.ShapeDtypeStruct((BATCH, NCLS), jnp.float32),
    )(embs, W_i2h, W_h2o, b_h2o)


@jax.jit
def kernel(X, table, W_i2h, W_h2o, b_h2o):
    x_idx = X.reshape(BATCH * SEQ // WIN_IDX, WIN_IDX).astype(jnp.int32)
    embs = _sc_bag_mean(x_idx, table)
    return _tc_mlp(embs, W_i2h, W_h2o, b_h2o.reshape(1, NCLS))
